# baseline (speedup 1.0000x reference)
import functools

import numpy as np
import jax
import jax.numpy as jnp
from jax import lax
from jax.experimental import pallas as pl
from jax.experimental.pallas import tpu as pltpu
from jax.experimental.pallas import tpu_sc as plsc

_L = 16
_BIG = 1 << 22
_NOISE_CACHE = {}


def _gumbel_noise(B, A, K):
    if (B, A, K) not in _NOISE_CACHE:
        t = 0
        beta = 0.1 + 0.1 * np.sin(t / 1000)
        with jax.ensure_compile_time_eval():
            nkey = jax.random.key(42)
            _NOISE_CACHE[(B, A, K)] = np.stack(
                [np.asarray(beta * jax.random.gumbel(
                    jax.random.fold_in(nkey, j), (B, K), dtype=jnp.float32))
                 for j in range(A)], axis=1)
    return _NOISE_CACHE[(B, A, K)]


def _make_places_body(P):
    def body(x_ref, places_ref):
        for p in range(P):
            h = x_ref[0, pl.ds(2 * p, 1), :]
            v = x_ref[0, pl.ds(2 * p + 1, 1), :]
            places_ref[0, p] = lax.dot_general(
                h, v, (((0,), (0,)), ((), ())),
                precision=lax.Precision.HIGHEST,
                preferred_element_type=jnp.float32)
    return body


def _make_sc_sampled(B, A, K, n_workers):
    P = A // 2
    BP = B * P
    maps_per_w = BP // n_workers
    batches_per_w = B // n_workers
    nchunks = K // _L

    H = K // 2

    def body(x_ref, lp_ref, zeros_ref, out_ref, xbuf, lpbuf, zbuf, sems):
        wid = lax.axis_index("c") * 16 + lax.axis_index("s")
        base_map = wid * maps_per_w
        pltpu.sync_copy(zeros_ref, zbuf)
        pltpu.sync_copy(x_ref.at[pl.ds(wid * batches_per_w, batches_per_w)],
                        xbuf)
        pltpu.sync_copy(lp_ref.at[pl.ds(wid * batches_per_w, batches_per_w)],
                        lpbuf)

        lane = jax.lax.iota(jnp.int32, _L)
        zvec = jnp.zeros((_L,), jnp.float32)

        def winners(bl, row):
            mvec = jnp.full((_L,), -jnp.inf, jnp.float32)
            for c in range(nchunks):
                mvec = jnp.maximum(mvec, lpbuf[bl, row, pl.ds(c * _L, _L)])
            m = jnp.max(mvec)
            c1 = jnp.full((_L,), _BIG, jnp.int32)
            for c in range(nchunks):
                x = lpbuf[bl, row, pl.ds(c * _L, _L)]
                idx = lane + (c * _L)
                c1 = jnp.minimum(c1, jnp.where(x == m, idx, _BIG))
            i1 = jnp.min(c1)
            c2 = jnp.full((_L,), _BIG, jnp.int32)
            for c in range(nchunks):
                x = lpbuf[bl, row, pl.ds(c * _L, _L)]
                idx = lane + (c * _L)
                c2 = jnp.minimum(
                    c2, jnp.where((x == m) & (idx != i1), idx, _BIG))
            i2 = jnp.min(c2)
            return i1, i2

        pending = [None, None]

        def drain(slot):
            if pending[slot] is not None:
                cpy, idxr, cjv, msk = pending[slot]
                cpy.wait()
                plsc.store_scatter(
                    zbuf, [jnp.full((_L,), slot, jnp.int32), idxr, cjv],
                    zvec, mask=msk)
                pending[slot] = None

        for mi in range(maps_per_w):
            bl, p = mi // P, mi % P
            i1, i2 = winners(bl, 2 * p)
            j1, j2 = winners(bl, 2 * p + 1)
            i2e = jnp.where(i2 < _BIG, i2, i1)
            j2e = jnp.where(j2 < _BIG, j2, j1)
            ri = jnp.where(((lane >> 1) & 1) == 1, i2e, i1)
            cj = jnp.where((lane & 1) == 1, j2e, j1)
            blv = jnp.full((_L,), bl, jnp.int32)
            hval = plsc.load_gather(xbuf, [blv, jnp.full((_L,), 2 * p,
                                                         jnp.int32), ri])
            vval = plsc.load_gather(xbuf, [blv, jnp.full((_L,), 2 * p + 1,
                                                         jnp.int32), cj])
            vals = (hval * vval) * 100.0
            m = base_map + mi
            for hf in range(2):
                slot = hf
                drain(slot)
                msk = (ri >= hf * H) & (ri < (hf + 1) * H)
                idxr = jnp.clip(ri - hf * H, 0, H - 1)
                plsc.store_scatter(
                    zbuf, [jnp.full((_L,), slot, jnp.int32), idxr, cj],
                    vals, mask=msk)
                cpy = pltpu.make_async_copy(
                    zbuf.at[slot], out_ref.at[m, pl.ds(hf * H, H)],
                    sems.at[slot])
                cpy.start()
                pending[slot] = (cpy, idxr, cj, msk)
        drain(0)
        drain(1)

    mesh = plsc.VectorSubcoreMesh(core_axis_name="c", subcore_axis_name="s")
    return functools.partial(
        pl.kernel, body, mesh=mesh,
        compiler_params=pltpu.CompilerParams(needs_layout_passes=False),
        out_type=jax.ShapeDtypeStruct((BP, K, K), jnp.float32),
        scratch_types=[
            pltpu.VMEM((batches_per_w, A, K), jnp.float32),
            pltpu.VMEM((batches_per_w, A, K), jnp.float32),
            pltpu.VMEM((2, K // 2, K), jnp.float32),
            pltpu.SemaphoreType.DMA((2,)),
        ])()


def kernel(x_cat):
    B, A, K = x_cat.shape
    P = A // 2
    f32 = jnp.float32

    try:
        noise = jnp.asarray(_gumbel_noise(B, A, K))
    except Exception:
        t = 0
        beta = 0.1 + 0.1 * np.sin(t / 1000)
        nkey = jax.random.key(42)
        noise = jnp.stack(
            [beta * jax.random.gumbel(jax.random.fold_in(nkey, j), (B, K),
                                      dtype=f32)
             for j in range(A)], axis=1)
    lp = jnp.log(x_cat) + noise

    zeros = jnp.zeros((2, K // 2, K), f32)
    sampled = _make_sc_sampled(B, A, K, 32)(x_cat, lp, zeros)

    places = pl.pallas_call(
        _make_places_body(P),
        grid=(B,),
        in_specs=[pl.BlockSpec((1, A, K), lambda b: (b, 0, 0))],
        out_specs=pl.BlockSpec((1, P, K, K), lambda b: (b, 0, 0, 0)),
        out_shape=jax.ShapeDtypeStruct((B, P, K, K), f32),
        compiler_params=pltpu.CompilerParams(
            dimension_semantics=("parallel",)),
    )(x_cat)
    return (places, sampled.reshape(B, P, K, K))


# speedup vs baseline: 1.1397x; 1.1397x over previous
import functools

import numpy as np
import jax
import jax.numpy as jnp
from jax import lax
from jax.experimental import pallas as pl
from jax.experimental.pallas import tpu as pltpu
from jax.experimental.pallas import tpu_sc as plsc

_L = 16
_BIG = 1 << 22
_NOISE_CACHE = {}


def _gumbel_noise(B, A, K):
    if (B, A, K) not in _NOISE_CACHE:
        t = 0
        beta = 0.1 + 0.1 * np.sin(t / 1000)
        with jax.ensure_compile_time_eval():
            nkey = jax.random.key(42)
            _NOISE_CACHE[(B, A, K)] = np.stack(
                [np.asarray(beta * jax.random.gumbel(
                    jax.random.fold_in(nkey, j), (B, K), dtype=jnp.float32))
                 for j in range(A)], axis=1)
    return _NOISE_CACHE[(B, A, K)]


def _make_places_body(P, BB=1):
    def body(x_ref, places_ref):
        for b in range(BB):
            for p in range(P):
                h = x_ref[b, pl.ds(2 * p, 1), :]
                v = x_ref[b, pl.ds(2 * p + 1, 1), :]
                places_ref[b, p] = lax.dot_general(
                    h, v, (((0,), (0,)), ((), ())),
                    precision=lax.Precision.HIGHEST,
                    preferred_element_type=jnp.float32)
    return body


def _make_sc_sampled(B, A, K, n_workers):
    P = A // 2
    BP = B * P
    maps_per_w = BP // n_workers
    batches_per_w = B // n_workers
    nchunks = K // _L

    H = K // 2

    def body(x_ref, lp_ref, zeros_ref, out_ref, xbuf, lpbuf, zbuf, sems):
        wid = lax.axis_index("c") * 16 + lax.axis_index("s")
        base_map = wid * maps_per_w
        pltpu.sync_copy(zeros_ref, zbuf)
        pltpu.sync_copy(x_ref.at[pl.ds(wid * batches_per_w, batches_per_w)],
                        xbuf)
        pltpu.sync_copy(lp_ref.at[pl.ds(wid * batches_per_w, batches_per_w)],
                        lpbuf)

        lane = jax.lax.iota(jnp.int32, _L)
        zvec = jnp.zeros((_L,), jnp.float32)

        def winners(bl, row):
            mvec = jnp.full((_L,), -jnp.inf, jnp.float32)
            for c in range(nchunks):
                mvec = jnp.maximum(mvec, lpbuf[bl, row, pl.ds(c * _L, _L)])
            m = jnp.max(mvec)
            c1 = jnp.full((_L,), _BIG, jnp.int32)
            for c in range(nchunks):
                x = lpbuf[bl, row, pl.ds(c * _L, _L)]
                idx = lane + (c * _L)
                c1 = jnp.minimum(c1, jnp.where(x == m, idx, _BIG))
            i1 = jnp.min(c1)
            c2 = jnp.full((_L,), _BIG, jnp.int32)
            for c in range(nchunks):
                x = lpbuf[bl, row, pl.ds(c * _L, _L)]
                idx = lane + (c * _L)
                c2 = jnp.minimum(
                    c2, jnp.where((x == m) & (idx != i1), idx, _BIG))
            i2 = jnp.min(c2)
            return i1, i2

        pending = [None, None]

        def drain(slot):
            if pending[slot] is not None:
                cpy, idxr, cjv, msk = pending[slot]
                cpy.wait()
                plsc.store_scatter(
                    zbuf, [jnp.full((_L,), slot, jnp.int32), idxr, cjv],
                    zvec, mask=msk)
                pending[slot] = None

        for mi in range(maps_per_w):
            bl, p = mi // P, mi % P
            i1, i2 = winners(bl, 2 * p)
            j1, j2 = winners(bl, 2 * p + 1)
            i2e = jnp.where(i2 < _BIG, i2, i1)
            j2e = jnp.where(j2 < _BIG, j2, j1)
            ri = jnp.where(((lane >> 1) & 1) == 1, i2e, i1)
            cj = jnp.where((lane & 1) == 1, j2e, j1)
            blv = jnp.full((_L,), bl, jnp.int32)
            hval = plsc.load_gather(xbuf, [blv, jnp.full((_L,), 2 * p,
                                                         jnp.int32), ri])
            vval = plsc.load_gather(xbuf, [blv, jnp.full((_L,), 2 * p + 1,
                                                         jnp.int32), cj])
            vals = (hval * vval) * 100.0
            m = base_map + mi
            for hf in range(2):
                slot = hf
                drain(slot)
                msk = (ri >= hf * H) & (ri < (hf + 1) * H)
                idxr = jnp.clip(ri - hf * H, 0, H - 1)
                plsc.store_scatter(
                    zbuf, [jnp.full((_L,), slot, jnp.int32), idxr, cj],
                    vals, mask=msk)
                cpy = pltpu.make_async_copy(
                    zbuf.at[slot], out_ref.at[m, pl.ds(hf * H, H)],
                    sems.at[slot])
                cpy.start()
                pending[slot] = (cpy, idxr, cj, msk)
        drain(0)
        drain(1)

    mesh = plsc.VectorSubcoreMesh(core_axis_name="c", subcore_axis_name="s")
    return functools.partial(
        pl.kernel, body, mesh=mesh,
        compiler_params=pltpu.CompilerParams(needs_layout_passes=False),
        out_type=jax.ShapeDtypeStruct((BP, K, K), jnp.float32),
        scratch_types=[
            pltpu.VMEM((batches_per_w, A, K), jnp.float32),
            pltpu.VMEM((batches_per_w, A, K), jnp.float32),
            pltpu.VMEM((2, K // 2, K), jnp.float32),
            pltpu.SemaphoreType.DMA((2,)),
        ])()


def kernel(x_cat):
    B, A, K = x_cat.shape
    P = A // 2
    f32 = jnp.float32

    try:
        noise = jnp.asarray(_gumbel_noise(B, A, K))
    except Exception:
        t = 0
        beta = 0.1 + 0.1 * np.sin(t / 1000)
        nkey = jax.random.key(42)
        noise = jnp.stack(
            [beta * jax.random.gumbel(jax.random.fold_in(nkey, j), (B, K),
                                      dtype=f32)
             for j in range(A)], axis=1)
    lp = jnp.log(x_cat) + noise

    zeros = jnp.zeros((2, K // 2, K), f32)
    sampled = _make_sc_sampled(B, A, K, 32)(x_cat, lp, zeros)

    places = pl.pallas_call(
        _make_places_body(P, 2),
        grid=(B // 2,),
        in_specs=[pl.BlockSpec((2, A, K), lambda b: (b, 0, 0))],
        out_specs=pl.BlockSpec((2, P, K, K), lambda b: (b, 0, 0, 0)),
        out_shape=jax.ShapeDtypeStruct((B, P, K, K), f32),
        compiler_params=pltpu.CompilerParams(
            dimension_semantics=("parallel",)),
    )(x_cat)
    return (places, sampled.reshape(B, P, K, K))


# speedup vs baseline: 1.2336x; 1.0824x over previous
import functools

import numpy as np
import jax
import jax.numpy as jnp
from jax import lax
from jax.experimental import pallas as pl
from jax.experimental.pallas import tpu as pltpu
from jax.experimental.pallas import tpu_sc as plsc

_L = 16
_BIG = 1 << 22
_NOISE_CACHE = {}


def _gumbel_noise(B, A, K):
    if (B, A, K) not in _NOISE_CACHE:
        t = 0
        beta = 0.1 + 0.1 * np.sin(t / 1000)
        with jax.ensure_compile_time_eval():
            nkey = jax.random.key(42)
            _NOISE_CACHE[(B, A, K)] = np.stack(
                [np.asarray(beta * jax.random.gumbel(
                    jax.random.fold_in(nkey, j), (B, K), dtype=jnp.float32))
                 for j in range(A)], axis=1)
    return _NOISE_CACHE[(B, A, K)]


def _make_places_body(P, BB=1):
    def body(x_ref, places_ref):
        for b in range(BB):
            for p in range(P):
                h = x_ref[b, pl.ds(2 * p, 1), :]
                v = x_ref[b, pl.ds(2 * p + 1, 1), :]
                places_ref[b, p] = lax.dot_general(
                    h, v, (((0,), (0,)), ((), ())),
                    precision=lax.Precision.HIGHEST,
                    preferred_element_type=jnp.float32)
    return body


def _make_sc_sampled(B, A, K, n_workers):
    P = A // 2
    BP = B * P
    maps_per_w = BP // n_workers
    batches_per_w = B // n_workers
    nchunks = K // _L

    H = K // 2

    def body(x_ref, lp_ref, zeros_ref, out_ref, xbuf, lpbuf, zbuf, sems):
        wid = lax.axis_index("c") * 16 + lax.axis_index("s")
        base_map = wid * maps_per_w
        pltpu.sync_copy(zeros_ref, zbuf)
        pltpu.sync_copy(x_ref.at[pl.ds(wid * batches_per_w, batches_per_w)],
                        xbuf)
        pltpu.sync_copy(lp_ref.at[pl.ds(wid * batches_per_w, batches_per_w)],
                        lpbuf)

        lane = jax.lax.iota(jnp.int32, _L)
        zvec = jnp.zeros((_L,), jnp.float32)

        def winners(bl, row):
            mvec = jnp.full((_L,), -jnp.inf, jnp.float32)
            for c in range(nchunks):
                mvec = jnp.maximum(mvec, lpbuf[bl, row, pl.ds(c * _L, _L)])
            m = jnp.max(mvec)
            c1 = jnp.full((_L,), _BIG, jnp.int32)
            for c in range(nchunks):
                x = lpbuf[bl, row, pl.ds(c * _L, _L)]
                idx = lane + (c * _L)
                c1 = jnp.minimum(c1, jnp.where(x == m, idx, _BIG))
            i1 = jnp.min(c1)
            c2 = jnp.full((_L,), _BIG, jnp.int32)
            for c in range(nchunks):
                x = lpbuf[bl, row, pl.ds(c * _L, _L)]
                idx = lane + (c * _L)
                c2 = jnp.minimum(
                    c2, jnp.where((x == m) & (idx != i1), idx, _BIG))
            i2 = jnp.min(c2)
            return i1, i2

        pending = [None, None]

        def drain(slot):
            if pending[slot] is not None:
                cpy, idxr, cjv, msk = pending[slot]
                cpy.wait()
                plsc.store_scatter(
                    zbuf, [jnp.full((_L,), slot, jnp.int32), idxr, cjv],
                    zvec, mask=msk)
                pending[slot] = None

        for mi in range(maps_per_w):
            bl, p = mi // P, mi % P
            i1, i2 = winners(bl, 2 * p)
            j1, j2 = winners(bl, 2 * p + 1)
            i2e = jnp.where(i2 < _BIG, i2, i1)
            j2e = jnp.where(j2 < _BIG, j2, j1)
            ri = jnp.where(((lane >> 1) & 1) == 1, i2e, i1)
            cj = jnp.where((lane & 1) == 1, j2e, j1)
            blv = jnp.full((_L,), bl, jnp.int32)
            hval = plsc.load_gather(xbuf, [blv, jnp.full((_L,), 2 * p,
                                                         jnp.int32), ri])
            vval = plsc.load_gather(xbuf, [blv, jnp.full((_L,), 2 * p + 1,
                                                         jnp.int32), cj])
            vals = (hval * vval) * 100.0
            m = base_map + mi
            for hf in range(2):
                slot = hf
                drain(slot)
                msk = (ri >= hf * H) & (ri < (hf + 1) * H)
                idxr = jnp.clip(ri - hf * H, 0, H - 1)
                plsc.store_scatter(
                    zbuf, [jnp.full((_L,), slot, jnp.int32), idxr, cj],
                    vals, mask=msk)
                cpy = pltpu.make_async_copy(
                    zbuf.at[slot], out_ref.at[m, pl.ds(hf * H, H)],
                    sems.at[slot])
                cpy.start()
                pending[slot] = (cpy, idxr, cj, msk)
        drain(0)
        drain(1)

    mesh = plsc.VectorSubcoreMesh(core_axis_name="c", subcore_axis_name="s")
    return functools.partial(
        pl.kernel, body, mesh=mesh,
        compiler_params=pltpu.CompilerParams(needs_layout_passes=False),
        out_type=jax.ShapeDtypeStruct((BP, K, K), jnp.float32),
        scratch_types=[
            pltpu.VMEM((batches_per_w, A, K), jnp.float32),
            pltpu.VMEM((batches_per_w, A, K), jnp.float32),
            pltpu.VMEM((2, K // 2, K), jnp.float32),
            pltpu.SemaphoreType.DMA((2,)),
        ])()


def kernel(x_cat):
    B, A, K = x_cat.shape
    P = A // 2
    f32 = jnp.float32

    try:
        noise = jnp.asarray(_gumbel_noise(B, A, K))
    except Exception:
        t = 0
        beta = 0.1 + 0.1 * np.sin(t / 1000)
        nkey = jax.random.key(42)
        noise = jnp.stack(
            [beta * jax.random.gumbel(jax.random.fold_in(nkey, j), (B, K),
                                      dtype=f32)
             for j in range(A)], axis=1)
    lp = jnp.log(x_cat) + noise

    zeros = jnp.zeros((2, K // 2, K), f32)
    sampled = _make_sc_sampled(B, A, K, 32)(x_cat, lp, zeros)

    places = pl.pallas_call(
        _make_places_body(P, 4),
        grid=(B // 4,),
        in_specs=[pl.BlockSpec((4, A, K), lambda b: (b, 0, 0))],
        out_specs=pl.BlockSpec((4, P, K, K), lambda b: (b, 0, 0, 0)),
        out_shape=jax.ShapeDtypeStruct((B, P, K, K), f32),
        compiler_params=pltpu.CompilerParams(
            dimension_semantics=("parallel",)),
    )(x_cat)
    return (places, sampled.reshape(B, P, K, K))


# speedup vs baseline: 1.2668x; 1.0269x over previous
import functools

import numpy as np
import jax
import jax.numpy as jnp
from jax import lax
from jax.experimental import pallas as pl
from jax.experimental.pallas import tpu as pltpu
from jax.experimental.pallas import tpu_sc as plsc

_L = 16
_BIG = 1 << 22
_NOISE_CACHE = {}


def _gumbel_noise(B, A, K):
    if (B, A, K) not in _NOISE_CACHE:
        t = 0
        beta = 0.1 + 0.1 * np.sin(t / 1000)
        with jax.ensure_compile_time_eval():
            nkey = jax.random.key(42)
            _NOISE_CACHE[(B, A, K)] = np.stack(
                [np.asarray(beta * jax.random.gumbel(
                    jax.random.fold_in(nkey, j), (B, K), dtype=jnp.float32))
                 for j in range(A)], axis=1)
    return _NOISE_CACHE[(B, A, K)]


def _make_places_body(P, BB=1):
    def body(x_ref, places_ref):
        for b in range(BB):
            for p in range(P):
                h = x_ref[b, pl.ds(2 * p, 1), :]
                v = x_ref[b, pl.ds(2 * p + 1, 1), :]
                places_ref[b, p] = lax.dot_general(
                    h, v, (((0,), (0,)), ((), ())),
                    precision=lax.Precision.HIGHEST,
                    preferred_element_type=jnp.float32)
    return body


def _make_sc_sampled(B, A, K, n_workers):
    P = A // 2
    BP = B * P
    maps_per_w = BP // n_workers
    batches_per_w = B // n_workers
    nchunks = K // _L

    H = K // 2

    def body(x_ref, lp_ref, zeros_ref, out_ref, xbuf, lpbuf, zbuf, sems):
        wid = lax.axis_index("c") * 16 + lax.axis_index("s")
        base_map = wid * maps_per_w
        pltpu.sync_copy(zeros_ref, zbuf)
        pltpu.sync_copy(x_ref.at[pl.ds(wid * batches_per_w, batches_per_w)],
                        xbuf)
        pltpu.sync_copy(lp_ref.at[pl.ds(wid * batches_per_w, batches_per_w)],
                        lpbuf)

        lane = jax.lax.iota(jnp.int32, _L)
        zvec = jnp.zeros((_L,), jnp.float32)

        def winners(bl, row):
            mvec = jnp.full((_L,), -jnp.inf, jnp.float32)
            for c in range(nchunks):
                mvec = jnp.maximum(mvec, lpbuf[bl, row, pl.ds(c * _L, _L)])
            m = jnp.max(mvec)
            c1 = jnp.full((_L,), _BIG, jnp.int32)
            for c in range(nchunks):
                x = lpbuf[bl, row, pl.ds(c * _L, _L)]
                idx = lane + (c * _L)
                c1 = jnp.minimum(c1, jnp.where(x == m, idx, _BIG))
            i1 = jnp.min(c1)
            c2 = jnp.full((_L,), _BIG, jnp.int32)
            for c in range(nchunks):
                x = lpbuf[bl, row, pl.ds(c * _L, _L)]
                idx = lane + (c * _L)
                c2 = jnp.minimum(
                    c2, jnp.where((x == m) & (idx != i1), idx, _BIG))
            i2 = jnp.min(c2)
            return i1, i2

        pending = [None, None]

        def drain(slot):
            if pending[slot] is not None:
                cpy, idxr, cjv, msk = pending[slot]
                cpy.wait()
                plsc.store_scatter(
                    zbuf, [jnp.full((_L,), slot, jnp.int32), idxr, cjv],
                    zvec, mask=msk)
                pending[slot] = None

        for mi in range(maps_per_w):
            bl, p = mi // P, mi % P
            i1, i2 = winners(bl, 2 * p)
            j1, j2 = winners(bl, 2 * p + 1)
            i2e = jnp.where(i2 < _BIG, i2, i1)
            j2e = jnp.where(j2 < _BIG, j2, j1)
            ri = jnp.where(((lane >> 1) & 1) == 1, i2e, i1)
            cj = jnp.where((lane & 1) == 1, j2e, j1)
            blv = jnp.full((_L,), bl, jnp.int32)
            hval = plsc.load_gather(xbuf, [blv, jnp.full((_L,), 2 * p,
                                                         jnp.int32), ri])
            vval = plsc.load_gather(xbuf, [blv, jnp.full((_L,), 2 * p + 1,
                                                         jnp.int32), cj])
            vals = (hval * vval) * 100.0
            m = base_map + mi
            for hf in range(2):
                slot = hf
                drain(slot)
                msk = (ri >= hf * H) & (ri < (hf + 1) * H)
                idxr = jnp.clip(ri - hf * H, 0, H - 1)
                plsc.store_scatter(
                    zbuf, [jnp.full((_L,), slot, jnp.int32), idxr, cj],
                    vals, mask=msk)
                cpy = pltpu.make_async_copy(
                    zbuf.at[slot], out_ref.at[m, pl.ds(hf * H, H)],
                    sems.at[slot])
                cpy.start()
                pending[slot] = (cpy, idxr, cj, msk)
        drain(0)
        drain(1)

    mesh = plsc.VectorSubcoreMesh(core_axis_name="c", subcore_axis_name="s")
    return functools.partial(
        pl.kernel, body, mesh=mesh,
        compiler_params=pltpu.CompilerParams(needs_layout_passes=False),
        out_type=jax.ShapeDtypeStruct((BP, K, K), jnp.float32),
        scratch_types=[
            pltpu.VMEM((batches_per_w, A, K), jnp.float32),
            pltpu.VMEM((batches_per_w, A, K), jnp.float32),
            pltpu.VMEM((2, K // 2, K), jnp.float32),
            pltpu.SemaphoreType.DMA((2,)),
        ])()


def kernel(x_cat):
    B, A, K = x_cat.shape
    P = A // 2
    f32 = jnp.float32

    try:
        noise = jnp.asarray(_gumbel_noise(B, A, K))
    except Exception:
        t = 0
        beta = 0.1 + 0.1 * np.sin(t / 1000)
        nkey = jax.random.key(42)
        noise = jnp.stack(
            [beta * jax.random.gumbel(jax.random.fold_in(nkey, j), (B, K),
                                      dtype=f32)
             for j in range(A)], axis=1)
    lp = jnp.log(x_cat) + noise

    zeros = jnp.zeros((2, K // 2, K), f32)
    sampled = _make_sc_sampled(B, A, K, 32)(x_cat, lp, zeros)

    places = pl.pallas_call(
        _make_places_body(P, 8),
        grid=(B // 8,),
        in_specs=[pl.BlockSpec((8, A, K), lambda b: (b, 0, 0))],
        out_specs=pl.BlockSpec((8, P, K, K), lambda b: (b, 0, 0, 0)),
        out_shape=jax.ShapeDtypeStruct((B, P, K, K), f32),
        compiler_params=pltpu.CompilerParams(
            dimension_semantics=("parallel",)),
    )(x_cat)
    return (places, sampled.reshape(B, P, K, K))
